# Initial kernel scaffold; baseline (speedup 1.0000x reference)
import jax
import jax.numpy as jnp
from jax.experimental import pallas as pl


def kernel(x):
    raise NotImplementedError("write your pallas kernel here")


# speedup vs baseline: 13.3882x; 13.3882x over previous
import functools

import jax
import jax.numpy as jnp
from jax import lax
from jax.experimental import pallas as pl
from jax.experimental.pallas import tpu as pltpu
from jax.experimental.pallas import tpu_sc as plsc

B = 128
K = 4096
DPC = 8
N = K * DPC

NW = 32
ROWS_PER_W = B // NW
CHUNKS_PER_ROW = 4
CHUNK = K // CHUNKS_PER_ROW
WIN_WORDS = 7 * CHUNK + 16
VAL_WORDS = 8 * CHUNK


def _dk_body(x_hbm, out_hbm, win_v, val_v, out_v):
    wid = lax.axis_index("s") * 2 + lax.axis_index("c")

    def zero_body(i, carry):
        out_v[pl.ds(i * 16, 16)] = jnp.zeros((16,), jnp.float32)
        return carry

    def win_body(i, carry):
        kloc = i * 16 + lax.iota(jnp.int32, 16)
        base7 = kloc * 7
        m = plsc.load_gather(win_v, [base7])
        w = jnp.zeros((16,), jnp.int32)
        for j in range(1, DPC):
            cj = plsc.load_gather(win_v, [base7 + j])
            gt = cj > m
            m = jnp.where(gt, cj, m)
            w = jnp.where(gt, j, w)
        oidx = kloc * 8 + w
        vals = plsc.load_gather(val_v, [oidx])
        plsc.store_scatter(out_v, [oidx], vals)
        return carry

    for r in range(ROWS_PER_W):
        row = wid * ROWS_PER_W + r
        rowbase = row * N
        for c in range(CHUNKS_PER_ROW):
            k0 = c * CHUNK
            pltpu.sync_copy(x_hbm.at[pl.ds(rowbase + 7 * k0, WIN_WORDS)], win_v)
            pltpu.sync_copy(x_hbm.at[pl.ds(rowbase + 8 * k0, VAL_WORDS)], val_v)
            lax.fori_loop(0, VAL_WORDS // 16, zero_body, 0)
            lax.fori_loop(0, CHUNK // 16, win_body, 0)
            pltpu.sync_copy(out_v, out_hbm.at[pl.ds(rowbase + 8 * k0, VAL_WORDS)])


@jax.jit
def kernel(x):
    mesh = plsc.VectorSubcoreMesh(core_axis_name="c", subcore_axis_name="s")
    run = functools.partial(
        pl.kernel,
        mesh=mesh,
        out_type=jax.ShapeDtypeStruct((B * N,), jnp.float32),
        compiler_params=pltpu.CompilerParams(needs_layout_passes=False),
        scratch_types=[
            pltpu.VMEM((WIN_WORDS,), jnp.float32),
            pltpu.VMEM((VAL_WORDS,), jnp.float32),
            pltpu.VMEM((VAL_WORDS,), jnp.float32),
        ],
    )(_dk_body)
    out = run(x.reshape(-1))
    return out.reshape(B, N)


# speedup vs baseline: 24.2469x; 1.8111x over previous
import functools

import jax
import jax.numpy as jnp
from jax import lax
from jax.experimental import pallas as pl
from jax.experimental.pallas import tpu as pltpu
from jax.experimental.pallas import tpu_sc as plsc

B = 128
K = 4096
DPC = 8
N = K * DPC

NW = 32
ROWS_PER_W = B // NW
HALF = N // 2
KHALF = K // 2


def _dk_body(x_hbm, out_hbm, x0, x1, o0, o1, sx0, sx1, so0, so1):
    wid = lax.axis_index("s") * 2 + lax.axis_index("c")
    xbufs = (x0, x1)
    obufs = (o0, o1)
    xsems = (sx0, sx1)
    osems = (so0, so1)

    def in_copy(r, b):
        row = wid * ROWS_PER_W + r
        return pltpu.async_copy(x_hbm.at[pl.ds(row * N, N)], xbufs[b], xsems[b])

    def make_win(xbuf, obuf, h):
        def win_body(i, carry):
            zeros = jnp.zeros((16,), jnp.float32)
            for v in range(8):
                obuf[pl.ds(i * 128 + v * 16, 16)] = zeros
            kloc = h * KHALF + i * 16 + lax.iota(jnp.int32, 16)
            base7 = kloc * 7
            m = plsc.load_gather(xbuf, [base7])
            w = jnp.zeros((16,), jnp.int32)
            for j in range(1, DPC):
                cj = plsc.load_gather(xbuf, [base7 + j])
                gt = cj > m
                m = jnp.where(gt, cj, m)
                w = jnp.where(gt, j, w)
            oidx = kloc * 8 + w
            vals = plsc.load_gather(xbuf, [oidx])
            plsc.store_scatter(obuf, [oidx - h * HALF], vals)
            return carry
        return win_body

    in_handles = [None] * ROWS_PER_W
    out_handles = [None] * (2 * ROWS_PER_W)
    in_handles[0] = in_copy(0, 0)
    for r in range(ROWS_PER_W):
        xb = r % 2
        if r + 1 < ROWS_PER_W:
            in_handles[r + 1] = in_copy(r + 1, (r + 1) % 2)
        row = wid * ROWS_PER_W + r
        for h in range(2):
            ob = h
            ci = 2 * r + h
            if ci >= 2:
                out_handles[ci - 2].wait()
            if h == 0:
                in_handles[r].wait()
            lax.fori_loop(0, KHALF // 16, make_win(xbufs[xb], obufs[ob], h), 0)
            out_handles[ci] = pltpu.async_copy(
                obufs[ob], out_hbm.at[pl.ds(row * N + h * HALF, HALF)], osems[ob])
    out_handles[-2].wait()
    out_handles[-1].wait()


@jax.jit
def kernel(x):
    mesh = plsc.VectorSubcoreMesh(core_axis_name="c", subcore_axis_name="s")
    run = functools.partial(
        pl.kernel,
        mesh=mesh,
        out_type=jax.ShapeDtypeStruct((B * N,), jnp.float32),
        compiler_params=pltpu.CompilerParams(needs_layout_passes=False),
        scratch_types=[
            pltpu.VMEM((N,), jnp.float32),
            pltpu.VMEM((N,), jnp.float32),
            pltpu.VMEM((HALF,), jnp.float32),
            pltpu.VMEM((HALF,), jnp.float32),
            pltpu.SemaphoreType.DMA,
            pltpu.SemaphoreType.DMA,
            pltpu.SemaphoreType.DMA,
            pltpu.SemaphoreType.DMA,
        ],
    )(_dk_body)
    out = run(x.reshape(-1))
    return out.reshape(B, N)


# speedup vs baseline: 28.4912x; 1.1750x over previous
import functools

import jax
import jax.numpy as jnp
from jax import lax
from jax.experimental import pallas as pl
from jax.experimental.pallas import tpu as pltpu
from jax.experimental.pallas import tpu_sc as plsc

B = 128
K = 4096
DPC = 8
N = K * DPC

NW = 32
ROWS_PER_W = B // NW
HALF = N // 2
KHALF = K // 2


def _dk_body(x_hbm, out_hbm, x0, x1, o0, o1, sx0, sx1, so0, so1):
    wid = lax.axis_index("s") * 2 + lax.axis_index("c")
    xbufs = (x0, x1)
    obufs = (o0, o1)
    xsems = (sx0, sx1)
    osems = (so0, so1)

    def in_copy(r, b):
        row = wid * ROWS_PER_W + r
        return pltpu.async_copy(x_hbm.at[pl.ds(row * N, N)], xbufs[b], xsems[b])

    def run_win(xbuf, obuf, h):
        @plsc.parallel_loop(0, KHALF // 16, unroll=4)
        def win_body(i):
            zeros = jnp.zeros((16,), jnp.float32)
            for v in range(8):
                obuf[pl.ds(i * 128 + v * 16, 16)] = zeros
            kloc = h * KHALF + i * 16 + lax.iota(jnp.int32, 16)
            base7 = kloc * 7
            m = plsc.load_gather(xbuf, [base7])
            w = jnp.zeros((16,), jnp.int32)
            for j in range(1, DPC):
                cj = plsc.load_gather(xbuf, [base7 + j])
                gt = cj > m
                m = jnp.where(gt, cj, m)
                w = jnp.where(gt, j, w)
            oidx = kloc * 8 + w
            vals = plsc.load_gather(xbuf, [oidx])
            plsc.store_scatter(obuf, [oidx - h * HALF], vals)

    in_handles = [None] * ROWS_PER_W
    out_handles = [None] * (2 * ROWS_PER_W)
    in_handles[0] = in_copy(0, 0)
    for r in range(ROWS_PER_W):
        xb = r % 2
        if r + 1 < ROWS_PER_W:
            in_handles[r + 1] = in_copy(r + 1, (r + 1) % 2)
        row = wid * ROWS_PER_W + r
        for h in range(2):
            ob = h
            ci = 2 * r + h
            if ci >= 2:
                out_handles[ci - 2].wait()
            if h == 0:
                in_handles[r].wait()
            run_win(xbufs[xb], obufs[ob], h)
            out_handles[ci] = pltpu.async_copy(
                obufs[ob], out_hbm.at[pl.ds(row * N + h * HALF, HALF)], osems[ob])
    out_handles[-2].wait()
    out_handles[-1].wait()


@jax.jit
def kernel(x):
    mesh = plsc.VectorSubcoreMesh(core_axis_name="c", subcore_axis_name="s")
    run = functools.partial(
        pl.kernel,
        mesh=mesh,
        out_type=jax.ShapeDtypeStruct((B * N,), jnp.float32),
        compiler_params=pltpu.CompilerParams(needs_layout_passes=False),
        scratch_types=[
            pltpu.VMEM((N,), jnp.float32),
            pltpu.VMEM((N,), jnp.float32),
            pltpu.VMEM((HALF,), jnp.float32),
            pltpu.VMEM((HALF,), jnp.float32),
            pltpu.SemaphoreType.DMA,
            pltpu.SemaphoreType.DMA,
            pltpu.SemaphoreType.DMA,
            pltpu.SemaphoreType.DMA,
        ],
    )(_dk_body)
    out = run(x.reshape(-1))
    return out.reshape(B, N)


# speedup vs baseline: 57.6035x; 2.0218x over previous
import functools

import jax
import jax.numpy as jnp
from jax import lax
from jax.experimental import pallas as pl
from jax.experimental.pallas import tpu as pltpu
from jax.experimental.pallas import tpu_sc as plsc

B = 128
K = 4096
DPC = 8
N = K * DPC

NW = 32
ROWS_PER_W = B // NW
HALF = N // 2
KHALF = K // 2


def _dk_body(x_hbm, out_hbm, x0, x1, o0, o1, sx0, sx1, so0, so1):
    wid = lax.axis_index("s") * 2 + lax.axis_index("c")
    xbufs = (x0, x1)
    obufs = (o0, o1)
    xsems = (sx0, sx1)
    osems = (so0, so1)

    def in_copy(r, b):
        row = wid * ROWS_PER_W + r
        return pltpu.async_copy(x_hbm.at[row], xbufs[b], xsems[b])

    def run_win(xbuf, obuf, h):
        @plsc.parallel_loop(0, KHALF // 16, unroll=4)
        def win_body(i):
            zeros = jnp.zeros((16,), jnp.float32)
            for v in range(8):
                obuf[pl.ds(i * 128 + v * 16, 16)] = zeros
            kloc = h * KHALF + i * 16 + lax.iota(jnp.int32, 16)
            base7 = kloc * 7
            m = plsc.load_gather(xbuf, [base7])
            w = jnp.zeros((16,), jnp.int32)
            for j in range(1, DPC):
                cj = plsc.load_gather(xbuf, [base7 + j])
                gt = cj > m
                m = jnp.where(gt, cj, m)
                w = jnp.where(gt, j, w)
            oidx = kloc * 8 + w
            vals = plsc.load_gather(xbuf, [oidx])
            plsc.store_scatter(obuf, [oidx - h * HALF], vals)

    in_handles = [None] * ROWS_PER_W
    out_handles = [None] * (2 * ROWS_PER_W)
    in_handles[0] = in_copy(0, 0)
    for r in range(ROWS_PER_W):
        xb = r % 2
        if r + 1 < ROWS_PER_W:
            in_handles[r + 1] = in_copy(r + 1, (r + 1) % 2)
        row = wid * ROWS_PER_W + r
        for h in range(2):
            ob = h
            ci = 2 * r + h
            if ci >= 2:
                out_handles[ci - 2].wait()
            if h == 0:
                in_handles[r].wait()
            run_win(xbufs[xb], obufs[ob], h)
            out_handles[ci] = pltpu.async_copy(
                obufs[ob], out_hbm.at[row, pl.ds(h * HALF, HALF)], osems[ob])
    out_handles[-2].wait()
    out_handles[-1].wait()


@jax.jit
def kernel(x):
    mesh = plsc.VectorSubcoreMesh(core_axis_name="c", subcore_axis_name="s")
    run = functools.partial(
        pl.kernel,
        mesh=mesh,
        out_type=jax.ShapeDtypeStruct((B, N), jnp.float32),
        compiler_params=pltpu.CompilerParams(needs_layout_passes=False),
        scratch_types=[
            pltpu.VMEM((N,), jnp.float32),
            pltpu.VMEM((N,), jnp.float32),
            pltpu.VMEM((HALF,), jnp.float32),
            pltpu.VMEM((HALF,), jnp.float32),
            pltpu.SemaphoreType.DMA,
            pltpu.SemaphoreType.DMA,
            pltpu.SemaphoreType.DMA,
            pltpu.SemaphoreType.DMA,
        ],
    )(_dk_body)
    return run(x)


# speedup vs baseline: 59.4537x; 1.0321x over previous
import functools

import jax
import jax.numpy as jnp
from jax import lax
from jax.experimental import pallas as pl
from jax.experimental.pallas import tpu as pltpu
from jax.experimental.pallas import tpu_sc as plsc

B = 128
K = 4096
DPC = 8
N = K * DPC

NW = 32
ROWS_PER_W = B // NW
HALF = N // 2
KHALF = K // 2
XOFF = (7 * KHALF // 8) * 8
XWORDS = N - XOFF
UNITS = 2 * ROWS_PER_W
ITERS = KHALF // 16


def _dk_body(x_hbm, out_hbm, x0, x1, o0, o1, sx0, sx1, so0, so1):
    wid = lax.axis_index("s") * 2 + lax.axis_index("c")
    row0 = wid * ROWS_PER_W
    xbufs = (x0, x1)
    obufs = (o0, o1)
    xsems = (sx0, sx1)
    osems = (so0, so1)
    xstart = (0, XOFF)
    voff = (0, 16384 - XOFF)

    def in_copy(row, b):
        return pltpu.async_copy(
            x_hbm.at[row, pl.ds(xstart[b], XWORDS)], xbufs[b], xsems[b])

    def out_copy(row, b):
        return pltpu.async_copy(
            obufs[b], out_hbm.at[row, pl.ds(b * HALF, HALF)], osems[b])

    def run_win(b):
        xbuf = xbufs[b]
        obuf = obufs[b]
        vo = voff[b]
        b7_0 = lax.iota(jnp.int32, 16) * 7
        b8_0 = lax.iota(jnp.int32, 16) * 8

        @plsc.parallel_loop(0, ITERS, unroll=4, carry=(b7_0, b8_0))
        def win_body(i, c):
            b7, b8 = c
            zeros = jnp.zeros((16,), jnp.float32)
            for v in range(8):
                obuf[pl.ds(i * 128 + v * 16, 16)] = zeros
            m = plsc.load_gather(xbuf, [b7])
            w = jnp.zeros((16,), jnp.int32)
            for j in range(1, DPC):
                cj = plsc.load_gather(xbuf, [b7 + j])
                gt = cj > m
                m = jnp.where(gt, cj, m)
                w = jnp.where(gt, j, w)
            oidx = b8 + w
            vals = plsc.load_gather(xbuf, [oidx + vo])
            plsc.store_scatter(obuf, [oidx], vals)
            return (b7 + 112, b8 + 128)

    in_copy(row0, 0)
    in_copy(row0, 1)

    def ring_body(g, carry):
        row = row0 + g
        for b in range(2):
            pltpu.make_async_copy(
                x_hbm.at[row, pl.ds(xstart[b], XWORDS)], xbufs[b], xsems[b]
            ).wait()

            @pl.when(g >= 1)
            def _():
                pltpu.make_async_copy(
                    obufs[b], out_hbm.at[row - 1, pl.ds(b * HALF, HALF)],
                    osems[b]).wait()

            run_win(b)
            out_copy(row, b)

            @pl.when(g < ROWS_PER_W - 1)
            def _():
                in_copy(row + 1, b)
        return carry

    lax.fori_loop(0, ROWS_PER_W, ring_body, 0)
    last = row0 + ROWS_PER_W - 1
    for b in range(2):
        pltpu.make_async_copy(
            obufs[b], out_hbm.at[last, pl.ds(b * HALF, HALF)], osems[b]).wait()


@jax.jit
def kernel(x):
    mesh = plsc.VectorSubcoreMesh(core_axis_name="c", subcore_axis_name="s")
    run = functools.partial(
        pl.kernel,
        mesh=mesh,
        out_type=jax.ShapeDtypeStruct((B, N), jnp.float32),
        compiler_params=pltpu.CompilerParams(needs_layout_passes=False),
        scratch_types=[
            pltpu.VMEM((XWORDS,), jnp.float32),
            pltpu.VMEM((XWORDS,), jnp.float32),
            pltpu.VMEM((HALF,), jnp.float32),
            pltpu.VMEM((HALF,), jnp.float32),
            pltpu.SemaphoreType.DMA,
            pltpu.SemaphoreType.DMA,
            pltpu.SemaphoreType.DMA,
            pltpu.SemaphoreType.DMA,
        ],
    )(_dk_body)
    return run(x)


# speedup vs baseline: 59.9801x; 1.0089x over previous
import functools

import jax
import jax.numpy as jnp
from jax import lax
from jax.experimental import pallas as pl
from jax.experimental.pallas import tpu as pltpu
from jax.experimental.pallas import tpu_sc as plsc

B = 128
K = 4096
DPC = 8
N = K * DPC

NW = 32
ROWS_PER_W = B // NW
HALF = N // 2
KHALF = K // 2
XOFF = (7 * KHALF // 8) * 8
XW0 = HALF
XW1 = N - XOFF
UNITS = 2 * ROWS_PER_W
ITERS = KHALF // 16


def _dk_body(x_hbm, out_hbm, x0, x1, o0, o1, sx0, sx1, so0, so1):
    wid = lax.axis_index("s") * 2 + lax.axis_index("c")
    row0 = wid * ROWS_PER_W
    xbufs = (x0, x1)
    obufs = (o0, o1)
    xsems = (sx0, sx1)
    osems = (so0, so1)
    xstart = (0, XOFF)
    xwords = (XW0, XW1)
    voff = (0, 16384 - XOFF)

    def in_copy(row, b):
        return pltpu.async_copy(
            x_hbm.at[row, pl.ds(xstart[b], xwords[b])], xbufs[b], xsems[b])

    def out_copy(row, b):
        return pltpu.async_copy(
            obufs[b], out_hbm.at[row, pl.ds(b * HALF, HALF)], osems[b])

    def run_win(b):
        xbuf = xbufs[b]
        obuf = obufs[b]
        vo = voff[b]
        b7_0 = lax.iota(jnp.int32, 16) * 7
        b8_0 = lax.iota(jnp.int32, 16) * 8

        @plsc.parallel_loop(0, ITERS, unroll=4, carry=(b7_0, b8_0))
        def win_body(i, c):
            b7, b8 = c
            zeros = jnp.zeros((16,), jnp.float32)
            for v in range(8):
                obuf[pl.ds(i * 128 + v * 16, 16)] = zeros
            m = plsc.load_gather(xbuf, [b7])
            w = jnp.zeros((16,), jnp.int32)
            for j in range(1, DPC):
                cj = plsc.load_gather(xbuf, [b7 + j])
                gt = cj > m
                m = jnp.where(gt, cj, m)
                w = jnp.where(gt, j, w)
            oidx = b8 + w
            vals = plsc.load_gather(xbuf, [oidx + vo])
            plsc.store_scatter(obuf, [oidx], vals)
            return (b7 + 112, b8 + 128)

    in_copy(row0, 0)
    in_copy(row0, 1)

    def ring_body(g, carry):
        row = row0 + g
        for b in range(2):
            pltpu.make_async_copy(
                x_hbm.at[row, pl.ds(xstart[b], xwords[b])], xbufs[b], xsems[b]
            ).wait()

            @pl.when(g >= 1)
            def _():
                pltpu.make_async_copy(
                    obufs[b], out_hbm.at[row - 1, pl.ds(b * HALF, HALF)],
                    osems[b]).wait()

            run_win(b)
            out_copy(row, b)

            @pl.when(g < ROWS_PER_W - 1)
            def _():
                in_copy(row + 1, b)
        return carry

    lax.fori_loop(0, ROWS_PER_W, ring_body, 0)
    last = row0 + ROWS_PER_W - 1
    for b in range(2):
        pltpu.make_async_copy(
            obufs[b], out_hbm.at[last, pl.ds(b * HALF, HALF)], osems[b]).wait()


@jax.jit
def kernel(x):
    mesh = plsc.VectorSubcoreMesh(core_axis_name="c", subcore_axis_name="s")
    run = functools.partial(
        pl.kernel,
        mesh=mesh,
        out_type=jax.ShapeDtypeStruct((B, N), jnp.float32),
        compiler_params=pltpu.CompilerParams(needs_layout_passes=False),
        scratch_types=[
            pltpu.VMEM((XW0,), jnp.float32),
            pltpu.VMEM((XW1,), jnp.float32),
            pltpu.VMEM((HALF,), jnp.float32),
            pltpu.VMEM((HALF,), jnp.float32),
            pltpu.SemaphoreType.DMA,
            pltpu.SemaphoreType.DMA,
            pltpu.SemaphoreType.DMA,
            pltpu.SemaphoreType.DMA,
        ],
    )(_dk_body)
    return run(x)
